# baseline (speedup 1.0000x reference)
import functools

import jax
import jax.numpy as jnp
from jax import lax
from jax.experimental import pallas as pl
from jax.experimental.pallas import tpu as pltpu
from jax.experimental.pallas import tpu_sc as plsc

_LANES = 16


def _make_lookup(B: int, D: int):
    info = plsc.get_sparse_core_info()
    NS = info.num_subcores
    NW = info.num_cores * NS
    assert B % (8 * NW) == 0
    b_per_w = B // NW
    mesh = plsc.VectorSubcoreMesh(core_axis_name="c", subcore_axis_name="s")

    @functools.partial(
        pl.kernel,
        mesh=mesh,
        out_type=jax.ShapeDtypeStruct((B, D), jnp.float32),
        scratch_types=[
            pltpu.VMEM((b_per_w,), jnp.int32),
            pltpu.VMEM((b_per_w, D), jnp.float32),
            pltpu.VMEM_SHARED((2 * NS, D), jnp.float32),
            pltpu.SemaphoreType.DMA,
            pltpu.SemaphoreType.DMA,
            pltpu.SemaphoreType.DMA,
        ],
    )
    def lookup(
        idx_hbm, table_hbm, out_hbm, idx_v, rows_v, shared_tab, isem, gsem, wsem
    ):
        sid = lax.axis_index("s")
        wid = sid * info.num_cores + lax.axis_index("c")
        base = wid * b_per_w

        icopy = pltpu.make_async_copy(
            idx_hbm.at[pl.ds(base, b_per_w)], idx_v, isem
        )
        icopy.start()
        pltpu.sync_copy(table_hbm, shared_tab.at[pl.ds(2 * sid, 2)])
        icopy.wait()

        off = jnp.full((_LANES,), 2 * sid, dtype=jnp.int32)
        for j in range(b_per_w // _LANES):
            sl = pl.ds(j * _LANES, _LANES)
            idx_v[sl] = idx_v[sl] + off

        n_pieces = 4
        rp = b_per_w // n_pieces
        gathers = [
            pltpu.make_async_copy(
                shared_tab.at[idx_v.at[pl.ds(k * rp, rp)]],
                rows_v.at[pl.ds(k * rp, rp)],
                gsem,
            )
            for k in range(n_pieces)
        ]
        writes = [
            pltpu.make_async_copy(
                rows_v.at[pl.ds(k * rp, rp)],
                out_hbm.at[pl.ds(base + k * rp, rp)],
                wsem,
            )
            for k in range(n_pieces)
        ]
        gathers[0].start()
        for k in range(n_pieces):
            if k + 1 < n_pieces:
                gathers[k + 1].start()
            gathers[k].wait()
            writes[k].start()
        for k in range(n_pieces):
            writes[k].wait()

    return lookup


def kernel(indices, table):
    B = indices.shape[0]
    D = table.shape[1]
    return _make_lookup(B, D)(indices, table)


# speedup vs baseline: 1.0013x; 1.0013x over previous
import functools

import jax
import jax.numpy as jnp
from jax import lax
from jax.experimental import pallas as pl
from jax.experimental.pallas import tpu as pltpu
from jax.experimental.pallas import tpu_sc as plsc

_LANES = 16


def _make_lookup(B: int, D: int):
    info = plsc.get_sparse_core_info()
    NS = info.num_subcores
    NW = info.num_cores * NS
    assert B % (8 * NW) == 0
    b_per_w = B // NW
    mesh = plsc.VectorSubcoreMesh(core_axis_name="c", subcore_axis_name="s")

    @functools.partial(
        pl.kernel,
        mesh=mesh,
        out_type=jax.ShapeDtypeStruct((B, D), jnp.float32),
        scratch_types=[
            pltpu.VMEM((b_per_w,), jnp.int32),
            pltpu.VMEM((b_per_w, D), jnp.float32),
            pltpu.VMEM_SHARED((2 * NS, D), jnp.float32),
            pltpu.SemaphoreType.DMA,
            pltpu.SemaphoreType.DMA,
            pltpu.SemaphoreType.DMA,
        ],
    )
    def lookup(
        idx_hbm, table_hbm, out_hbm, idx_v, rows_v, shared_tab, isem, gsem, wsem
    ):
        sid = lax.axis_index("s")
        wid = sid * info.num_cores + lax.axis_index("c")
        base = wid * b_per_w

        icopy = pltpu.make_async_copy(
            idx_hbm.at[pl.ds(base, b_per_w)], idx_v, isem
        )
        icopy.start()
        tcopy = pltpu.make_async_copy(
            table_hbm, shared_tab.at[pl.ds(2 * sid, 2)], gsem
        )
        tcopy.start()
        icopy.wait()

        off = jnp.full((_LANES,), 2 * sid, dtype=jnp.int32)
        for j in range(b_per_w // _LANES):
            sl = pl.ds(j * _LANES, _LANES)
            idx_v[sl] = idx_v[sl] + off
        tcopy.wait()

        if b_per_w == 512:
            bounds = [0, 64, 192, 320, 448, b_per_w]
        else:
            q = b_per_w // 4
            bounds = [0, q, 2 * q, 3 * q, b_per_w]
        pieces = list(zip(bounds[:-1], bounds[1:]))
        gathers = [
            pltpu.make_async_copy(
                shared_tab.at[idx_v.at[pl.ds(lo, hi - lo)]],
                rows_v.at[pl.ds(lo, hi - lo)],
                gsem,
            )
            for lo, hi in pieces
        ]
        writes = [
            pltpu.make_async_copy(
                rows_v.at[pl.ds(lo, hi - lo)],
                out_hbm.at[pl.ds(base + lo, hi - lo)],
                wsem,
            )
            for lo, hi in pieces
        ]
        gathers[0].start()
        for k in range(len(pieces)):
            if k + 1 < len(pieces):
                gathers[k + 1].start()
            gathers[k].wait()
            writes[k].start()
        for k in range(len(pieces)):
            writes[k].wait()

    return lookup


def kernel(indices, table):
    B = indices.shape[0]
    D = table.shape[1]
    return _make_lookup(B, D)(indices, table)
